# Initial kernel scaffold; baseline (speedup 1.0000x reference)
import jax
import jax.numpy as jnp
from jax.experimental import pallas as pl


def kernel(tensor, text_emb, W1, b1, W2, b2, Ws, bs, Wt, bt, Wo, bo, gate_param):
    raise NotImplementedError("write your pallas kernel here")


# speedup vs baseline: 1.0098x; 1.0098x over previous
import jax
import jax.numpy as jnp
from jax import lax
from jax.experimental import pallas as pl
from jax.experimental.pallas import tpu as pltpu


def _ln_rows(x):
    m = jnp.mean(x, axis=-1, keepdims=True)
    v = jnp.mean((x - m) ** 2, axis=-1, keepdims=True)
    return (x - m) / jnp.sqrt(v + 1e-5)


def _text_body(text_ref, wt_ref, bt_ref, gate_ref, out_ref):
    t = lax.dot_general(text_ref[...], wt_ref[...],
                        (((1,), (1,)), ((), ())),
                        preferred_element_type=jnp.float32)
    t = jax.nn.relu(t + bt_ref[...]) * 0.1
    tn = _ln_rows(t)
    gate = jax.nn.sigmoid(gate_ref[0, 0])
    out_ref[...] = gate * tn


def _mlp_body(sp_ref, ws_ref, bs_ref, tg_ref, wo_ref, bo_ref, fused_ref):
    sp = sp_ref[0]
    fs = lax.dot_general(sp, ws_ref[...], (((1,), (1,)), ((), ())),
                         preferred_element_type=jnp.float32) + bs_ref[...]
    fsn = _ln_rows(fs) + tg_ref[0]
    fo = lax.dot_general(fsn, wo_ref[...], (((1,), (1,)), ((), ())),
                         preferred_element_type=jnp.float32) + bo_ref[...]
    nrm = jnp.sqrt(jnp.sum(fo * fo, axis=1, keepdims=True))
    spn = jnp.sqrt(jnp.sum(sp * sp, axis=1, keepdims=True))
    fused_ref[0] = fo / jnp.maximum(nrm, 1e-12) * spn


def kernel(tensor, text_emb, W1, b1, W2, b2, Ws, bs, Wt, bt, Wo, bo, gate_param):
    B, C, H, Wd = tensor.shape
    HW = H * Wd
    K = max(1, int(HW * 0.5))
    embed_dim = Ws.shape[0]

    feat_flat = jnp.transpose(tensor.reshape(B, C, HW), (0, 2, 1))
    dn = ('NCHW', 'OIHW', 'NCHW')
    x = lax.conv_general_dilated(tensor, W1, (1, 1), [(0, 0), (0, 0)],
                                 dimension_numbers=dn) + b1.reshape(1, -1, 1, 1)
    x1 = lax.conv_general_dilated(x, W2, (1, 1), [(2, 2), (2, 2)],
                                  rhs_dilation=(2, 2), dimension_numbers=dn) + b2.reshape(1, -1, 1, 1)
    density_map = jax.nn.relu(x1 + x)
    density_flat = density_map.reshape(B, -1)
    _, topk_idx = lax.top_k(density_flat, K)
    sparse_feat = jnp.take_along_axis(feat_flat, topk_idx[:, :, None], axis=1)

    tg = pl.pallas_call(
        _text_body,
        out_shape=jax.ShapeDtypeStruct((B, embed_dim), jnp.float32),
    )(text_emb, Wt, bt.reshape(1, -1), gate_param.reshape(1, 1))

    RB = 512
    fused = pl.pallas_call(
        _mlp_body,
        grid=(B, K // RB),
        in_specs=[
            pl.BlockSpec((1, RB, C), lambda b, r: (b, r, 0)),
            pl.BlockSpec((embed_dim, C), lambda b, r: (0, 0)),
            pl.BlockSpec((1, embed_dim), lambda b, r: (0, 0)),
            pl.BlockSpec((1, 1, embed_dim), lambda b, r: (b, 0, 0)),
            pl.BlockSpec((C, embed_dim), lambda b, r: (0, 0)),
            pl.BlockSpec((1, C), lambda b, r: (0, 0)),
        ],
        out_specs=pl.BlockSpec((1, RB, C), lambda b, r: (b, r, 0)),
        out_shape=jax.ShapeDtypeStruct((B, K, C), jnp.float32),
    )(sparse_feat, Ws, bs.reshape(1, -1), tg.reshape(B, 1, embed_dim), Wo, bo.reshape(1, -1))

    tensor_flat = feat_flat.at[jnp.arange(B)[:, None], topk_idx].set(fused)
    out = jnp.transpose(tensor_flat, (0, 2, 1)).reshape(B, C, H, Wd)
    return out, density_map, topk_idx, fused
